# Initial kernel scaffold; baseline (speedup 1.0000x reference)
import jax
import jax.numpy as jnp
from jax.experimental import pallas as pl


def kernel(x, edge_index, W, b):
    raise NotImplementedError("write your pallas kernel here")


# speedup vs baseline: 11.0911x; 11.0911x over previous
import functools

import jax
import jax.numpy as jnp
from jax import lax
from jax.experimental import pallas as pl
from jax.experimental.pallas import tpu as pltpu
from jax.experimental.pallas import tpu_sc as plsc

N = 10000
E = 320000
D = 128
NC = 2
NS = 16
NW = NC * NS
K = 64
CH = 160
EPAD = NW * K * CH
NPAD = NS * 640

_MESH = dict(core_axis_name="c", subcore_axis_name="s")


@functools.partial(
    pl.kernel,
    out_type=jax.ShapeDtypeStruct((NC, NPAD), jnp.float32),
    mesh=plsc.VectorSubcoreMesh(**_MESH),
    scratch_types=[
        pltpu.VMEM((CH, K), jnp.int32),
        pltpu.VMEM((K,), jnp.float32),
        pltpu.VMEM((640,), jnp.float32),
        pltpu.VMEM_SHARED((NPAD,), jnp.float32),
    ],
)
def _deg_sc(dst_hbm, out_hbm, dst_v, ones_v, zb_v, acc):
    cid = lax.axis_index("c")
    sid = lax.axis_index("s")
    wid = sid * NC + cid

    for i in range(K // 16):
        ones_v[pl.ds(i * 16, 16)] = jnp.ones((16,), jnp.float32)

    def zf(i, _):
        zb_v[pl.ds(i * 16, 16)] = jnp.zeros((16,), jnp.float32)
        return ()
    lax.fori_loop(0, 40, zf, ())
    pltpu.sync_copy(zb_v, acc.at[pl.ds(sid * 640, 640)])

    pltpu.sync_copy(dst_hbm.at[wid], dst_v)
    plsc.subcore_barrier()

    def body(j, _):
        pltpu.sync_copy(ones_v, acc.at[dst_v.at[j]], add=True)
        return ()
    lax.fori_loop(0, CH, body, ())

    plsc.subcore_barrier()
    pltpu.sync_copy(acc.at[pl.ds(sid * 640, 640)],
                    out_hbm.at[cid, pl.ds(sid * 640, 640)])


@functools.partial(
    pl.kernel,
    out_type=jax.ShapeDtypeStruct((NC, NPAD, D), jnp.float32),
    mesh=plsc.VectorSubcoreMesh(**_MESH),
    scratch_types=[
        pltpu.VMEM((CH // 2, K), jnp.int32),
        pltpu.VMEM((CH // 2, K), jnp.int32),
        pltpu.VMEM((K, D), jnp.float32),
        pltpu.VMEM((K, D), jnp.float32),
        pltpu.VMEM((16, D), jnp.float32),
        pltpu.VMEM_SHARED((NPAD, D), jnp.float32),
        pltpu.SemaphoreType.DMA,
        pltpu.SemaphoreType.DMA,
    ],
)
def _msg_sc(h2_hbm, src_hbm, dst_hbm, out_hbm,
            src_v, dst_v, buf0, buf1, zrow_v, acc, sem0, sem1):
    cid = lax.axis_index("c")
    sid = lax.axis_index("s")
    wid = sid * NC + cid

    for r in range(16):
        for c in range(D // 16):
            zrow_v[r, pl.ds(c * 16, 16)] = jnp.zeros((16,), jnp.float32)

    def zero_acc(i, _):
        pltpu.sync_copy(zrow_v, acc.at[pl.ds(sid * 640 + i * 16, 16), :])
        return ()
    lax.fori_loop(0, 40, zero_acc, ())

    plsc.subcore_barrier()

    CHH = CH // 2
    for h in range(2):
        pltpu.sync_copy(src_hbm.at[wid, pl.ds(h * CHH, CHH)], src_v)
        pltpu.sync_copy(dst_hbm.at[wid, pl.ds(h * CHH, CHH)], dst_v)

        pltpu.async_copy(h2_hbm.at[src_v.at[0]], buf0, sem0)

        def body(t, _):
            j0 = 2 * t
            pltpu.async_copy(h2_hbm.at[src_v.at[j0 + 1]], buf1, sem1)
            pltpu.make_async_copy(h2_hbm.at[src_v.at[j0]], buf0, sem0).wait()
            pltpu.sync_copy(buf0, acc.at[dst_v.at[j0]], add=True)

            @pl.when(t < CHH // 2 - 1)
            def _():
                pltpu.async_copy(h2_hbm.at[src_v.at[j0 + 2]], buf0, sem0)

            pltpu.make_async_copy(h2_hbm.at[src_v.at[j0 + 1]], buf1, sem1).wait()
            pltpu.sync_copy(buf1, acc.at[dst_v.at[j0 + 1]], add=True)
            return ()
        lax.fori_loop(0, CHH // 2, body, ())

    plsc.subcore_barrier()
    pltpu.sync_copy(acc.at[pl.ds(sid * 640, 640), :],
                    out_hbm.at[cid, pl.ds(sid * 640, 640), :])


def _mm_fn(x_ref, w_ref, deg_ref, h2_ref):
    deg = deg_ref[:, 0] + deg_ref[:, 1]
    dinv = jnp.where(deg > 0, lax.rsqrt(jnp.maximum(deg, 1e-12)), 0.0)
    h = jnp.dot(x_ref[...], w_ref[...], preferred_element_type=jnp.float32)
    h2_ref[...] = h * dinv[:, None]


def _fin_fn(agg_ref, deg_ref, x_ref, b_ref, o_ref):
    deg = deg_ref[:, 0] + deg_ref[:, 1]
    dinv = jnp.where(deg > 0, lax.rsqrt(jnp.maximum(deg, 1e-12)), 0.0)
    agg = agg_ref[0] + agg_ref[1]
    o_ref[...] = jnp.maximum(agg * dinv[:, None] + b_ref[...], 0.0) + x_ref[...]


MBLK = 1000


def kernel(x, edge_index, W, b):
    src = edge_index[0].astype(jnp.int32)
    dst = edge_index[1].astype(jnp.int32)
    src3 = jnp.concatenate(
        [src, jnp.zeros((EPAD - E,), jnp.int32)]).reshape(NW, CH, K)
    dst3 = jnp.concatenate(
        [dst, jnp.full((EPAD - E,), N, jnp.int32)]).reshape(NW, CH, K)

    degp = _deg_sc(dst3)
    deg2 = jnp.transpose(degp[:, :N])

    h2 = pl.pallas_call(
        _mm_fn,
        grid=(N // MBLK,),
        in_specs=[
            pl.BlockSpec((MBLK, D), lambda i: (i, 0)),
            pl.BlockSpec((D, D), lambda i: (0, 0)),
            pl.BlockSpec((MBLK, NC), lambda i: (i, 0)),
        ],
        out_specs=pl.BlockSpec((MBLK, D), lambda i: (i, 0)),
        out_shape=jax.ShapeDtypeStruct((N, D), jnp.float32),
    )(x, W, deg2)

    aggp = _msg_sc(h2, src3, dst3)[:, :N, :]

    out = pl.pallas_call(
        _fin_fn,
        grid=(N // MBLK,),
        in_specs=[
            pl.BlockSpec((NC, MBLK, D), lambda i: (0, i, 0)),
            pl.BlockSpec((MBLK, NC), lambda i: (i, 0)),
            pl.BlockSpec((MBLK, D), lambda i: (i, 0)),
            pl.BlockSpec((1, D), lambda i: (0, 0)),
        ],
        out_specs=pl.BlockSpec((MBLK, D), lambda i: (i, 0)),
        out_shape=jax.ShapeDtypeStruct((N, D), jnp.float32),
    )(aggp, deg2, x, b.reshape(1, D))
    return out


# speedup vs baseline: 11.5560x; 1.0419x over previous
import functools

import jax
import jax.numpy as jnp
from jax import lax
from jax.experimental import pallas as pl
from jax.experimental.pallas import tpu as pltpu
from jax.experimental.pallas import tpu_sc as plsc

N = 10000
E = 320000
D = 128
NC = 2
NS = 16
NW = NC * NS
K = 64
SEG = 80
NSEG = 4
CHD = 160
EPAD = NSEG * NS * SEG * K
TOTCH = EPAD // K
NPAD = NS * 640

_MESH = dict(core_axis_name="c", subcore_axis_name="s")


@functools.partial(
    pl.kernel,
    out_type=jax.ShapeDtypeStruct((NC, NPAD), jnp.float32),
    mesh=plsc.VectorSubcoreMesh(**_MESH),
    scratch_types=[
        pltpu.VMEM((CHD, K), jnp.int32),
        pltpu.VMEM((K,), jnp.float32),
        pltpu.VMEM((640,), jnp.float32),
        pltpu.VMEM_SHARED((NPAD,), jnp.float32),
    ],
)
def _deg_sc(dst_hbm, out_hbm, dst_v, ones_v, zb_v, acc):
    cid = lax.axis_index("c")
    sid = lax.axis_index("s")
    wid = sid * NC + cid

    for i in range(K // 16):
        ones_v[pl.ds(i * 16, 16)] = jnp.ones((16,), jnp.float32)

    def zf(i, _):
        zb_v[pl.ds(i * 16, 16)] = jnp.zeros((16,), jnp.float32)
        return ()
    lax.fori_loop(0, 40, zf, ())
    pltpu.sync_copy(zb_v, acc.at[pl.ds(sid * 640, 640)])

    pltpu.sync_copy(dst_hbm.at[wid], dst_v)
    plsc.subcore_barrier()

    def body(j, _):
        pltpu.sync_copy(ones_v, acc.at[dst_v.at[j]], add=True)
        return ()
    lax.fori_loop(0, CHD, body, ())

    plsc.subcore_barrier()
    pltpu.sync_copy(acc.at[pl.ds(sid * 640, 640)],
                    out_hbm.at[cid, pl.ds(sid * 640, 640)])


@functools.partial(
    pl.kernel,
    out_type=jax.ShapeDtypeStruct((NC, NPAD, D), jnp.float32),
    mesh=plsc.VectorSubcoreMesh(**_MESH),
    scratch_types=[
        pltpu.VMEM((SEG, K), jnp.int32),
        pltpu.VMEM((SEG, K), jnp.int32),
        pltpu.VMEM((K, D), jnp.float32),
        pltpu.VMEM((K, D), jnp.float32),
        pltpu.VMEM((16, D), jnp.float32),
        pltpu.VMEM_SHARED((NPAD, D), jnp.float32),
        pltpu.SemaphoreType.DMA,
        pltpu.SemaphoreType.DMA,
    ],
)
def _msg_sc(h2_hbm, src_hbm, dst_hbm, out_hbm,
            src_v, dst_v, buf0, buf1, zrow_v, acc, sem0, sem1):
    cid = lax.axis_index("c")
    sid = lax.axis_index("s")

    for r in range(16):
        for c in range(D // 16):
            zrow_v[r, pl.ds(c * 16, 16)] = jnp.zeros((16,), jnp.float32)

    def zero_acc(i, _):
        pltpu.sync_copy(zrow_v, acc.at[pl.ds(sid * 640 + i * 16, 16), :])
        return ()
    lax.fori_loop(0, 40, zero_acc, ())
    plsc.subcore_barrier()

    def run_seg(base):
        pltpu.sync_copy(src_hbm.at[pl.ds(base, SEG)], src_v)
        pltpu.sync_copy(dst_hbm.at[pl.ds(base, SEG)], dst_v)
        pltpu.async_copy(h2_hbm.at[src_v.at[0]], buf0, sem0)

        def body(t, _):
            j0 = 2 * t
            pltpu.async_copy(h2_hbm.at[src_v.at[j0 + 1]], buf1, sem1)
            pltpu.make_async_copy(h2_hbm.at[src_v.at[j0]], buf0, sem0).wait()
            pltpu.sync_copy(buf0, acc.at[dst_v.at[j0]], add=True)

            @pl.when(t < SEG // 2 - 1)
            def _():
                pltpu.async_copy(h2_hbm.at[src_v.at[j0 + 2]], buf0, sem0)

            pltpu.make_async_copy(h2_hbm.at[src_v.at[j0 + 1]], buf1, sem1).wait()
            pltpu.sync_copy(buf1, acc.at[dst_v.at[j0 + 1]], add=True)
            return ()
        lax.fori_loop(0, SEG // 2, body, ())

    @pl.when(cid == 0)
    def _():
        for s in range(NSEG - 1):
            run_seg((s * NS + sid) * SEG)

    @pl.when(cid == 1)
    def _():
        run_seg(((NSEG - 1) * NS + sid) * SEG)

    plsc.subcore_barrier()
    pltpu.sync_copy(acc.at[pl.ds(sid * 640, 640), :],
                    out_hbm.at[cid, pl.ds(sid * 640, 640), :])


def _mm_fn(x_ref, w_ref, deg_ref, h2_ref):
    deg = deg_ref[:, 0] + deg_ref[:, 1]
    dinv = jnp.where(deg > 0, lax.rsqrt(jnp.maximum(deg, 1e-12)), 0.0)
    h = jnp.dot(x_ref[...], w_ref[...], preferred_element_type=jnp.float32)
    h2_ref[...] = h * dinv[:, None]


def _fin_fn(agg_ref, deg_ref, x_ref, b_ref, o_ref):
    deg = deg_ref[:, 0] + deg_ref[:, 1]
    dinv = jnp.where(deg > 0, lax.rsqrt(jnp.maximum(deg, 1e-12)), 0.0)
    agg = agg_ref[0] + agg_ref[1]
    o_ref[...] = jnp.maximum(agg * dinv[:, None] + b_ref[...], 0.0) + x_ref[...]


MBLK = 1000


def kernel(x, edge_index, W, b):
    src = edge_index[0].astype(jnp.int32)
    dst = edge_index[1].astype(jnp.int32)
    srcp = jnp.concatenate([src, jnp.zeros((EPAD - E,), jnp.int32)])
    dstp = jnp.concatenate([dst, jnp.full((EPAD - E,), N, jnp.int32)])

    degp = _deg_sc(dstp.reshape(NW, CHD, K))
    deg2 = jnp.transpose(degp[:, :N])

    h2 = pl.pallas_call(
        _mm_fn,
        grid=(N // MBLK,),
        in_specs=[
            pl.BlockSpec((MBLK, D), lambda i: (i, 0)),
            pl.BlockSpec((D, D), lambda i: (0, 0)),
            pl.BlockSpec((MBLK, NC), lambda i: (i, 0)),
        ],
        out_specs=pl.BlockSpec((MBLK, D), lambda i: (i, 0)),
        out_shape=jax.ShapeDtypeStruct((N, D), jnp.float32),
    )(x, W, deg2)

    aggp = _msg_sc(h2, srcp.reshape(TOTCH, K),
                   dstp.reshape(TOTCH, K))[:, :N, :]

    out = pl.pallas_call(
        _fin_fn,
        grid=(N // MBLK,),
        in_specs=[
            pl.BlockSpec((NC, MBLK, D), lambda i: (0, i, 0)),
            pl.BlockSpec((MBLK, NC), lambda i: (i, 0)),
            pl.BlockSpec((MBLK, D), lambda i: (i, 0)),
            pl.BlockSpec((1, D), lambda i: (0, 0)),
        ],
        out_specs=pl.BlockSpec((MBLK, D), lambda i: (i, 0)),
        out_shape=jax.ShapeDtypeStruct((N, D), jnp.float32),
    )(aggp, deg2, x, b.reshape(1, D))
    return out
